# speedup vs baseline: 4.7268x; 4.7268x over previous
import functools

import jax
import jax.numpy as jnp
from jax import lax
from jax.experimental import pallas as pl
from jax.experimental.pallas import tpu as pltpu
from jax.experimental.pallas import tpu_sc as plsc

D_MODEL = 128
SCALE = float(D_MODEL) ** 0.5
NC = 2
NS = 16
NW = NC * NS
CHUNK = 128


def _embed_call(ids_r, weight, n_chunks):
    mesh = plsc.VectorSubcoreMesh(core_axis_name="c", subcore_axis_name="s")
    b_total = NW * n_chunks * CHUNK

    @functools.partial(
        pl.kernel,
        mesh=mesh,
        out_type=jax.ShapeDtypeStruct((b_total, D_MODEL), jnp.float32),
        scratch_types=[
            pltpu.VMEM((n_chunks, CHUNK), jnp.int32),
            pltpu.VMEM((CHUNK, D_MODEL), jnp.float32),
            pltpu.SemaphoreType.DMA,
        ],
    )
    def _embed(ids_hbm, table_hbm, out_hbm, idx_v, rows_v, sem):
        wid = lax.axis_index("s") * NC + lax.axis_index("c")
        base = wid * (n_chunks * CHUNK)
        pltpu.sync_copy(ids_hbm.at[wid], idx_v)

        def chunk_body(g, carry):
            pltpu.async_copy(table_hbm.at[idx_v.at[g]], rows_v, sem).wait()

            def row_body(r, c2):
                for j in range(D_MODEL // 16):
                    sl = pl.ds(j * 16, 16)
                    rows_v[r, sl] = rows_v[r, sl] * SCALE
                return c2

            lax.fori_loop(0, CHUNK, row_body, 0)
            pltpu.sync_copy(rows_v, out_hbm.at[pl.ds(base + g * CHUNK, CHUNK)])
            return carry

        lax.fori_loop(0, n_chunks, chunk_body, 0)

    return _embed(ids_r, weight)


def kernel(ids, weight):
    b, s = ids.shape
    total = b * s
    n_chunks = total // (NW * CHUNK)
    assert total == NW * n_chunks * CHUNK
    ids_r = ids.reshape(NW, n_chunks, CHUNK)
    out = _embed_call(ids_r, weight, n_chunks)
    return out.reshape(b, s, D_MODEL)


# speedup vs baseline: 7.8405x; 1.6587x over previous
import functools

import jax
import jax.numpy as jnp
from jax import lax
from jax.experimental import pallas as pl
from jax.experimental.pallas import tpu as pltpu
from jax.experimental.pallas import tpu_sc as plsc

D_MODEL = 128
SCALE = float(D_MODEL) ** 0.5
NC = 2
NS = 16
NW = NC * NS
CHUNK = 128
N_BUF = 5
AHEAD = 3


def _embed_call(ids_r, weight, n_chunks):
    mesh = plsc.VectorSubcoreMesh(core_axis_name="c", subcore_axis_name="s")
    b_total = NW * n_chunks * CHUNK
    assert n_chunks % N_BUF == 0 and n_chunks // N_BUF >= 2

    @functools.partial(
        pl.kernel,
        mesh=mesh,
        out_type=jax.ShapeDtypeStruct((b_total, D_MODEL), jnp.float32),
        scratch_types=[
            pltpu.VMEM((n_chunks, CHUNK), jnp.int32),
            pltpu.VMEM((N_BUF, CHUNK, D_MODEL), jnp.float32),
            pltpu.SemaphoreType.DMA((N_BUF,)),
            pltpu.SemaphoreType.DMA((N_BUF,)),
        ],
    )
    def _embed(ids_hbm, table_hbm, out_hbm, idx_v, bufs, gsems, osems):
        wid = lax.axis_index("s") * NC + lax.axis_index("c")
        base = wid * (n_chunks * CHUNK)
        pltpu.sync_copy(ids_hbm.at[wid], idx_v)

        def g_start(g, b):
            pltpu.async_copy(table_hbm.at[idx_v.at[g]], bufs.at[b], gsems.at[b])

        def g_wait(g, b):
            pltpu.make_async_copy(
                table_hbm.at[idx_v.at[g]], bufs.at[b], gsems.at[b]).wait()

        def o_start(g, b):
            pltpu.async_copy(
                bufs.at[b], out_hbm.at[pl.ds(base + g * CHUNK, CHUNK)],
                osems.at[b])

        def o_wait(g, b):
            pltpu.make_async_copy(
                bufs.at[b], out_hbm.at[pl.ds(base + g * CHUNK, CHUNK)],
                osems.at[b]).wait()

        def scale(b):
            @plsc.parallel_loop(0, CHUNK, unroll=4)
            def _(r):
                for j in range(D_MODEL // 16):
                    sl = pl.ds(j * 16, 16)
                    bufs[b, r, sl] = bufs[b, r, sl] * SCALE

        for g in range(AHEAD):
            g_start(g, g)
        for g in range(N_BUF):
            b = g
            nxt = g + AHEAD
            tb = nxt % N_BUF
            if nxt >= N_BUF:
                o_wait(nxt - N_BUF, tb)
            g_start(nxt, tb)
            g_wait(g, b)
            scale(b)
            o_start(g, b)

        def outer(i, carry):
            g0 = i * N_BUF
            for b in range(N_BUF):
                g = g0 + b
                tb = (b + AHEAD) % N_BUF
                o_wait(g + AHEAD - N_BUF, tb)
                g_start(g + AHEAD, tb)
                g_wait(g, b)
                scale(b)
                o_start(g, b)
            return carry

        lax.fori_loop(1, n_chunks // N_BUF - 1, outer, 0)

        for g in range(n_chunks - N_BUF, n_chunks):
            b = g % N_BUF
            nxt = g + AHEAD
            if nxt < n_chunks:
                tb = nxt % N_BUF
                o_wait(nxt - N_BUF, tb)
                g_start(nxt, tb)
            g_wait(g, b)
            scale(b)
            o_start(g, b)
        for g in range(n_chunks - N_BUF, n_chunks):
            o_wait(g, g % N_BUF)

    return _embed(ids_r, weight)


def kernel(ids, weight):
    b, s = ids.shape
    total = b * s
    n_chunks = total // (NW * CHUNK)
    assert total == NW * n_chunks * CHUNK
    ids_r = ids.reshape(NW, n_chunks, CHUNK)
    out = _embed_call(ids_r, weight, n_chunks)
    return out.reshape(b, s, D_MODEL)
